# Initial kernel scaffold; baseline (speedup 1.0000x reference)
import jax
import jax.numpy as jnp
from jax.experimental import pallas as pl


def kernel(x, able, trigger):
    raise NotImplementedError("write your pallas kernel here")


# speedup vs baseline: 1295.5115x; 1295.5115x over previous
import jax
import jax.numpy as jnp
from jax import lax
from jax.experimental import pallas as pl
from jax.experimental.pallas import tpu as pltpu
from jax.experimental.pallas import tpu_sc as plsc

_NC = 2
_NS = 16
_NW = _NC * _NS
_LANES = 16

_ROWS = 50000
_COLS = 256
_NIDX = 20000
_TRIG = 64

_SEG = 1568
_MASK_PAD = _NW * _SEG


def _sc_mask_body(able_hbm, mask_hbm, idx_v, lmask):
    wid = lax.axis_index("s") * _NC + lax.axis_index("c")
    base = wid * _SEG

    def zero_body(i, _):
        lmask[pl.ds(i * _LANES, _LANES)] = jnp.zeros((_LANES,), jnp.float32)
        return _

    lax.fori_loop(0, _SEG // _LANES, zero_body, None)

    pltpu.sync_copy(able_hbm, idx_v)

    ones = jnp.ones((_LANES,), jnp.float32)

    def scan_body(i, _):
        v = idx_v[pl.ds(i * _LANES, _LANES)]
        local = v - base
        hit = (local >= 0) & (local < _SEG)
        plsc.store_scatter(lmask, [local], ones, mask=hit)
        return _

    lax.fori_loop(0, _NIDX // _LANES, scan_body, None)

    pltpu.sync_copy(lmask, mask_hbm.at[pl.ds(base, _SEG)])


def _sc_mask(able):
    mesh = plsc.VectorSubcoreMesh(core_axis_name="c", subcore_axis_name="s")
    return pl.kernel(
        _sc_mask_body,
        out_type=jax.ShapeDtypeStruct((_MASK_PAD,), jnp.float32),
        mesh=mesh,
        scratch_types=[
            pltpu.VMEM((_NIDX,), jnp.int32),
            pltpu.VMEM((_SEG,), jnp.float32),
        ],
        compiler_params=pltpu.CompilerParams(needs_layout_passes=False),
    )(able)


def _tc_body(x_ref, m_ref, t_ref, o_ref):
    xb = x_ref[...]
    mb = m_ref[...]
    tb = t_ref[...]
    col = lax.broadcasted_iota(jnp.int32, (_SEG, _COLS), 1)
    cond = (mb > 0.5) & (col < _TRIG)
    upd = jnp.minimum(xb + tb, 1.0)
    o_ref[...] = jnp.where(cond, upd, xb)


def _tc_apply(x, mask2, trow):
    grid = _MASK_PAD // _SEG
    return pl.pallas_call(
        _tc_body,
        grid=(grid,),
        in_specs=[
            pl.BlockSpec((_SEG, _COLS), lambda i: (i, 0)),
            pl.BlockSpec((_SEG, 1), lambda i: (i, 0)),
            pl.BlockSpec((1, _COLS), lambda i: (0, 0)),
        ],
        out_specs=pl.BlockSpec((_SEG, _COLS), lambda i: (i, 0)),
        out_shape=jax.ShapeDtypeStruct((_ROWS, _COLS), jnp.float32),
    )(x, mask2, trow)


def kernel(x, able, trigger):
    mask = _sc_mask(able.astype(jnp.int32))
    mask2 = mask.reshape(_MASK_PAD, 1)
    trow = jnp.concatenate(
        [trigger.astype(jnp.float32), jnp.zeros((_COLS - _TRIG,), jnp.float32)]
    ).reshape(1, _COLS)
    return _tc_apply(x, mask2, trow)
